# baseline (speedup 1.0000x reference)
import functools

import jax
import jax.numpy as jnp
from jax import lax
from jax.experimental import pallas as pl
from jax.experimental.pallas import tpu as pltpu
from jax.experimental.pallas import tpu_sc as plsc

_N = 100000
_K = 16
_C = 32
_H = 8
_M = 16
_NW = 32
_P = 32
_NCH = _N // _P
_ROWS = _P * _K
_GD_C = _P * _K * _H
_WH_C = _P * 8 * _M
_OUT_C = _P * _C * _M
_PAIRS = 49


def _pcf_body(feat_hbm, inds_hbm, gd_hbm, wa_hbm, wb_hbm, out_hbm,
              idx0, idx1, rows0, rows1, gd0, gd1, wa0, wa1, wb0, wb1,
              out0, out1, guided_t, wt_t, out_t,
              gsem0, gsem1, lsem0, lsem1, osem0, osem1):
    wid = lax.axis_index("s") * 2 + lax.axis_index("c")
    n_w = 97 + jnp.int32(wid < 21)
    start = wid * 97 + jnp.minimum(wid, 21)
    last = n_w - 1

    iota = lax.iota(jnp.int32, 16)
    pio16 = iota * 16
    pio128 = iota * 128

    def start_loads(g, idxv, rowsv, gdv, wav, wbv, gsem, lsem):
        pltpu.sync_copy(inds_hbm.at[g], idxv)
        for j in range(4):
            pltpu.async_copy(feat_hbm.at[idxv.at[j]],
                             rowsv.at[pl.ds(j * 128, 128)], gsem)
        pltpu.async_copy(gd_hbm.at[pl.ds(g * _GD_C, _GD_C)], gdv, lsem)
        pltpu.async_copy(wa_hbm.at[pl.ds(g * _WH_C, _WH_C)], wav, lsem)
        pltpu.async_copy(wb_hbm.at[pl.ds(g * _WH_C, _WH_C)], wbv, lsem)

    def wait_loads(g, idxv, rowsv, gdv, wav, wbv, gsem, lsem):
        for j in range(4):
            pltpu.make_async_copy(feat_hbm.at[idxv.at[j]],
                                  rowsv.at[pl.ds(j * 128, 128)], gsem).wait()
        pltpu.make_async_copy(gd_hbm.at[pl.ds(g * _GD_C, _GD_C)], gdv,
                              lsem).wait()
        pltpu.make_async_copy(wa_hbm.at[pl.ds(g * _WH_C, _WH_C)], wav,
                              lsem).wait()
        pltpu.make_async_copy(wb_hbm.at[pl.ds(g * _WH_C, _WH_C)], wbv,
                              lsem).wait()

    def out_store(g, outv, osem):
        return pltpu.make_async_copy(
            outv, out_hbm.at[pl.ds(g * _OUT_C, _OUT_C)], osem)

    def out_warm(g, outv, osem):
        return pltpu.make_async_copy(
            out_hbm.at[pl.ds(g * _OUT_C, _OUT_C)], outv, osem)

    def compute_chunk(rowsv, gdv, wav, wbv, outv):
        for grp in (0, 1):
            base_p = grp * 16

            def stage_a(kk, cr):
                gv = [plsc.load_gather(
                    gdv, [pio128 + (base_p * 128 + kk * 8 + hh)])
                    for hh in range(_H)]
                rowv = pio16 + (base_p * 16 + kk)
                for c in range(_C):
                    fv = plsc.load_gather(rowsv, [rowv, iota * 0 + c])
                    guided_t[pl.ds((kk * 32 + c) * 16, 16)] = fv * gv[c >> 2]
                return cr

            lax.fori_loop(0, _K, stage_a, 0)

            def stage_b(kk, cr, wv_ref, koff):
                for m in range(_M):
                    wv = plsc.load_gather(
                        wv_ref, [pio128 + (base_p * 128 + kk * 16 + m)])
                    wt_t[pl.ds(((kk + koff) * 16 + m) * 16, 16)] = wv
                return cr

            lax.fori_loop(0, 8, functools.partial(
                stage_b, wv_ref=wav, koff=0), 0)
            lax.fori_loop(0, 8, functools.partial(
                stage_b, wv_ref=wbv, koff=8), 0)

            def stage_c(tid, cr):
                ct = lax.shift_right_logical(tid, 2)
                mt = jnp.bitwise_and(tid, 3)
                accs = [None] * 32
                for kk in range(_K):
                    gbase = kk * 512 + ct * 128
                    wbase = kk * 256 + mt * 64
                    gl = [guided_t[pl.ds(gbase + j * 16, 16)]
                          for j in range(8)]
                    wl = [wt_t[pl.ds(wbase + jj * 16, 16)]
                          for jj in range(4)]
                    for j in range(8):
                        for jj in range(4):
                            t = gl[j] * wl[jj]
                            a = accs[j * 4 + jj]
                            accs[j * 4 + jj] = t if a is None else a + t
                obase = ct * 2048 + mt * 64
                for j in range(8):
                    for jj in range(4):
                        out_t[pl.ds(obase + j * 256 + jj * 16, 16)] = \
                            accs[j * 4 + jj]
                return cr

            lax.fori_loop(0, 16, stage_c, 0)

            def stage_d(p, cr):
                pp = base_p + p
                pbase = (lax.shift_right_logical(pp, 3) * 4096
                         + jnp.bitwise_and(pp, 7) * 128)
                for w_ in range(32):
                    j0 = w_ * 16
                    v = plsc.load_gather(out_t, [pio16 + (j0 * 16 + p)])
                    off = pbase + (j0 // 128) * 1024 + (j0 % 128)
                    outv[pl.ds(off, 16)] = v
                return cr

            lax.fori_loop(0, 16, stage_d, 0)

    slot = [
        (idx0, rows0, gd0, wa0, wb0, out0, gsem0, lsem0, osem0),
        (idx1, rows1, gd1, wa1, wb1, out1, gsem1, lsem1, osem1),
    ]

    for s in (0, 1):
        ix, rv, gv, wav_, wbv_, ov, gs, ls, os_ = slot[s]
        start_loads(start + s, ix, rv, gv, wav_, wbv_, gs, ls)
        out_warm(start + s, ov, os_).start()

    def pair_body(ii, carry):
        for s in (0, 1):
            ix, rv, gv, wav_, wbv_, ov, gs, ls, os_ = slot[s]
            cid = jnp.minimum(2 * ii + s, last)
            g = start + cid
            wait_loads(g, ix, rv, gv, wav_, wbv_, gs, ls)
            out_store(g, ov, os_).wait()
            compute_chunk(rv, gv, wav_, wbv_, ov)
            out_store(g, ov, os_).start()
            gp = start + jnp.minimum(2 * ii + s + 2, last)
            start_loads(gp, ix, rv, gv, wav_, wbv_, gs, ls)
        return carry

    lax.fori_loop(0, _PAIRS, pair_body, 0)

    for s in (0, 1):
        ix, rv, gv, wav_, wbv_, ov, gs, ls, os_ = slot[s]
        g = start + last
        wait_loads(g, ix, rv, gv, wav_, wbv_, gs, ls)
        out_store(g, ov, os_).wait()


@jax.jit
def _pcf_call(feat, inds, gd, wa, wb):
    mesh = plsc.VectorSubcoreMesh(core_axis_name="c", subcore_axis_name="s")
    kfn = functools.partial(
        pl.kernel,
        mesh=mesh,
        compiler_params=pltpu.CompilerParams(use_tc_tiling_on_sc=False,
                                             needs_layout_passes=False),
        out_type=jax.ShapeDtypeStruct((_N * _C * _M,), jnp.float32),
        scratch_types=[
            pltpu.VMEM((4, 128), jnp.int32),
            pltpu.VMEM((4, 128), jnp.int32),
            pltpu.VMEM((_ROWS, _C), jnp.float32),
            pltpu.VMEM((_ROWS, _C), jnp.float32),
            pltpu.VMEM((_GD_C,), jnp.float32),
            pltpu.VMEM((_GD_C,), jnp.float32),
            pltpu.VMEM((_WH_C,), jnp.float32),
            pltpu.VMEM((_WH_C,), jnp.float32),
            pltpu.VMEM((_WH_C,), jnp.float32),
            pltpu.VMEM((_WH_C,), jnp.float32),
            pltpu.VMEM((_OUT_C,), jnp.float32),
            pltpu.VMEM((_OUT_C,), jnp.float32),
            pltpu.VMEM((_K * _C * 16,), jnp.float32),
            pltpu.VMEM((_K * _M * 16,), jnp.float32),
            pltpu.VMEM((_C * _M * 16,), jnp.float32),
            pltpu.SemaphoreType.DMA,
            pltpu.SemaphoreType.DMA,
            pltpu.SemaphoreType.DMA,
            pltpu.SemaphoreType.DMA,
            pltpu.SemaphoreType.DMA,
            pltpu.SemaphoreType.DMA,
        ],
    )(_pcf_body)
    return kfn(feat, inds, gd, wa, wb)


def _t_mm(x2d):
    r = x2d.shape[0]
    eye = jnp.eye(r, dtype=jnp.float32)
    return lax.dot_general(x2d, eye, (((0,), (0,)), ((), ())),
                           precision=lax.Precision.HIGHEST)


def kernel(input_features, neighbor_inds, guidance, weightnet):
    b, n, c = input_features.shape
    k = neighbor_inds.shape[2]
    h = guidance.shape[3]
    m = weightnet.shape[3]
    assert (b, n, c, k, h, m) == (1, _N, _C, _K, _H, _M)
    feat = _t_mm(input_features[0].transpose(1, 0))
    inds_f = neighbor_inds[0].astype(jnp.float32).transpose(1, 0)
    inds = _t_mm(inds_f).astype(jnp.int32).reshape(_NCH, 4, 128)
    gd = _t_mm(guidance[0].transpose(1, 2, 0).reshape(k * h, n)).reshape(-1)
    w_t = weightnet[0].transpose(1, 2, 0).reshape(k * m, n)
    wa = _t_mm(w_t[: k * m // 2]).reshape(-1)
    wb = _t_mm(w_t[k * m // 2:]).reshape(-1)
    out = _pcf_call(feat, inds, gd, wa, wb)
    out = out.reshape(_N // 8, 4, 8, 128).transpose(0, 2, 1, 3)
    return out.reshape(b, n, c * m)
